# baseline (speedup 1.0000x reference)
import functools

import jax
import jax.numpy as jnp
from jax import lax
from jax.experimental import pallas as pl
from jax.experimental.pallas import tpu as pltpu
from jax.experimental.pallas import tpu_sc as plsc

_B = 4


def _sc_gather_bcast(idx, table):
    n_sel = idx.shape[0]
    n_pool, length, dim = table.shape
    info = plsc.get_sparse_core_info()
    lanes = info.num_lanes
    nw = info.num_cores * info.num_subcores
    assert n_sel % nw == 0
    per_w = n_sel // nw

    mesh = plsc.VectorSubcoreMesh(core_axis_name="c", subcore_axis_name="s")

    @functools.partial(
        pl.kernel,
        mesh=mesh,
        out_type=jax.ShapeDtypeStruct((_B, n_sel * length, 1, dim), jnp.float32),
        scratch_types=[
            pltpu.VMEM((n_sel + lanes,), jnp.int32),
            pltpu.VMEM((lanes,), jnp.int32),
            pltpu.VMEM((_B * per_w, length, dim), jnp.float32),
            pltpu.SemaphoreType.DMA,
        ],
    )
    def body(idx_hbm, table_hbm, out_hbm, idx_all, idx_mine, rows_v, gsem):
        wid = lax.axis_index("s") * info.num_cores + lax.axis_index("c")
        base = wid * per_w
        pltpu.sync_copy(idx_hbm, idx_all.at[pl.ds(0, n_sel)])
        window = idx_all[pl.ds(base, lanes)]
        idx_mine[...] = jnp.full((lanes,), window[0], jnp.int32)
        pltpu.async_copy(
            table_hbm.at[idx_mine.at[pl.ds(0, _B * per_w)]], rows_v, gsem
        ).wait()
        pltpu.sync_copy(rows_v, out_hbm.at[:, pl.ds(base * length, per_w * length), 0])

    return body(idx, table)


def kernel(indices, batch_size, prompts):
    del batch_size
    return _sc_gather_bcast(indices.astype(jnp.int32), prompts)
